# baseline (speedup 1.0000x reference)
import jax
import jax.numpy as jnp
from jax.experimental import pallas as pl
from jax.experimental.pallas import tpu as pltpu

N = 10000
NHID = 16
BM = 400
NM = N // BM
NBUF = 3


def _gcn_kernel(x_ref, adj_hbm, w_ref, b_ref, out_hbm, sup_ref):
    sup_ref[:, :] = jnp.dot(
        x_ref[:, :], w_ref[:, :], preferred_element_type=jnp.float32
    )

    def inner(adj_ref, out_ref):
        h = jnp.dot(
            adj_ref[:, :], sup_ref[:, :], preferred_element_type=jnp.float32
        )
        h = jax.nn.relu(h + b_ref[:, :])
        m = jnp.max(h, axis=1, keepdims=True)
        lse = jnp.log(jnp.sum(jnp.exp(h - m), axis=1, keepdims=True)) + m
        out_ref[:, :] = h - lse

    pltpu.emit_pipeline(
        inner,
        grid=(NM,),
        in_specs=[
            pl.BlockSpec(
                (BM, N),
                lambda i: (i, 0),
                pipeline_mode=pl.Buffered(buffer_count=NBUF),
            ),
        ],
        out_specs=[pl.BlockSpec((BM, NHID), lambda i: (i, 0))],
    )(adj_hbm, out_hbm)


@jax.jit
def _run(x, adj, W, b):
    return pl.pallas_call(
        _gcn_kernel,
        in_specs=[
            pl.BlockSpec(memory_space=pltpu.VMEM),
            pl.BlockSpec(memory_space=pl.ANY),
            pl.BlockSpec(memory_space=pltpu.VMEM),
            pl.BlockSpec(memory_space=pltpu.VMEM),
        ],
        out_specs=pl.BlockSpec(memory_space=pl.ANY),
        out_shape=jax.ShapeDtypeStruct((N, NHID), jnp.float32),
        scratch_shapes=[
            pltpu.VMEM((N, NHID), jnp.float32),
        ],
        compiler_params=pltpu.CompilerParams(
            vmem_limit_bytes=100 * 1024 * 1024,
        ),
    )(x, adj, W, b)


def kernel(x, adj, W, b):
    return _run(x, adj, W, b.reshape(1, -1))


# speedup vs baseline: 1.0283x; 1.0283x over previous
import jax
import jax.numpy as jnp
from jax.experimental import pallas as pl
from jax.experimental.pallas import tpu as pltpu

N = 10000
BM = 400
NM = N // BM


def _gcn_kernel(x_ref, adj_ref, w_ref, b_ref, out_ref, sup_ref):
    i = pl.program_id(0)

    @pl.when(i == 0)
    def _():
        sup_ref[:, :] = jnp.dot(
            x_ref[:, :], w_ref[:, :], preferred_element_type=jnp.float32
        )

    h = jnp.dot(adj_ref[:, :], sup_ref[:, :], preferred_element_type=jnp.float32)
    h = jax.nn.relu(h + b_ref[:, :])
    m = jnp.max(h, axis=1, keepdims=True)
    lse = jnp.log(jnp.sum(jnp.exp(h - m), axis=1, keepdims=True)) + m
    out_ref[pl.ds(i * BM, BM), :] = h - lse


@jax.jit
def _run(x, adj, W, b):
    nhid = W.shape[1]
    return pl.pallas_call(
        _gcn_kernel,
        grid=(NM,),
        in_specs=[
            pl.BlockSpec((N, x.shape[1]), lambda i: (0, 0)),
            pl.BlockSpec((BM, N), lambda i: (i, 0)),
            pl.BlockSpec((x.shape[1], nhid), lambda i: (0, 0)),
            pl.BlockSpec((1, nhid), lambda i: (0, 0)),
        ],
        out_specs=pl.BlockSpec((N, nhid), lambda i: (0, 0)),
        out_shape=jax.ShapeDtypeStruct((N, nhid), jnp.float32),
        scratch_shapes=[
            pltpu.VMEM((N, nhid), jnp.float32),
        ],
        compiler_params=pltpu.CompilerParams(
            vmem_limit_bytes=100 * 1024 * 1024,
        ),
    )(x, adj, W, b)


def kernel(x, adj, W, b):
    return _run(x, adj, W, b.reshape(1, -1))


# speedup vs baseline: 1.0406x; 1.0120x over previous
import jax
import jax.numpy as jnp
from jax.experimental import pallas as pl
from jax.experimental.pallas import tpu as pltpu

N = 10000
BM = 400
NM = N // BM


def _gcn_kernel(x_ref, adj_ref, w_ref, b_ref, out_ref, sup_ref):
    i = pl.program_id(0)

    @pl.when(i == 0)
    def _():
        sup_ref[:, :] = jnp.dot(
            x_ref[:, :], w_ref[:, :], preferred_element_type=jnp.float32
        )

    h = jnp.dot(adj_ref[:, :], sup_ref[:, :], preferred_element_type=jnp.float32)
    h = jax.nn.relu(h + b_ref[:, :])
    m = jnp.max(h, axis=1, keepdims=True)
    lse = jnp.log(jnp.sum(jnp.exp(h - m), axis=1, keepdims=True)) + m
    out_ref[:, :] = h - lse


@jax.jit
def _run(x, adj, W, b):
    nhid = W.shape[1]
    return pl.pallas_call(
        _gcn_kernel,
        grid=(NM,),
        in_specs=[
            pl.BlockSpec((N, x.shape[1]), lambda i: (0, 0)),
            pl.BlockSpec((BM, N), lambda i: (i, 0)),
            pl.BlockSpec((x.shape[1], nhid), lambda i: (0, 0)),
            pl.BlockSpec((1, nhid), lambda i: (0, 0)),
        ],
        out_specs=pl.BlockSpec((BM, nhid), lambda i: (i, 0)),
        out_shape=jax.ShapeDtypeStruct((N, nhid), jnp.float32),
        scratch_shapes=[
            pltpu.VMEM((N, nhid), jnp.float32),
        ],
        compiler_params=pltpu.CompilerParams(
            vmem_limit_bytes=100 * 1024 * 1024,
        ),
    )(x, adj, W, b)


def kernel(x, adj, W, b):
    return _run(x, adj, W, b.reshape(1, -1))
